# Initial kernel scaffold; baseline (speedup 1.0000x reference)
import jax
import jax.numpy as jnp
from jax.experimental import pallas as pl


def kernel(x, edge_index, edge_attr, Wq, Wk, Wv, We, be, Wo, bo):
    raise NotImplementedError("write your pallas kernel here")


# speedup vs baseline: 39.0328x; 39.0328x over previous
import functools

import jax
import jax.numpy as jnp
from jax import lax
from jax.experimental import pallas as pl
from jax.experimental.pallas import tpu as pltpu
from jax.experimental.pallas import tpu_sc as plsc

F32 = jnp.float32
NC = 2
NS = 16
NW = NC * NS
LANES = 16


_DG_DNUMS = lax.GatherDimensionNumbers(
    offset_dims=(), collapsed_slice_dims=(0,), start_index_map=(0,)
)


def _dg(x, idx):
    return lax.gather(x, idx.astype(jnp.int32)[:, None], _DG_DNUMS, (1,),
                      mode=lax.GatherScatterMode.PROMISE_IN_BOUNDS)


def _build_pass1(n, e, d, h, c):
    ew = e // NW
    dh = d // h
    g32 = 4 * h
    mesh = plsc.VectorSubcoreMesh(
        core_axis_name="c", subcore_axis_name="s", num_cores=NC, num_subcores=NS
    )
    scale = dh ** -0.5

    @functools.partial(
        pl.kernel,
        out_type=(
            jax.ShapeDtypeStruct((e * h,), F32),
            jax.ShapeDtypeStruct((NW, 2, LANES), F32),
        ),
        mesh=mesh,
        compiler_params=pltpu.CompilerParams(needs_layout_passes=False),
        scratch_types=(
            pltpu.VMEM((c,), jnp.int32),
            pltpu.VMEM((c,), jnp.int32),
            pltpu.VMEM((c, d), F32),
            pltpu.VMEM((c, d), F32),
            pltpu.VMEM((c, d), F32),
            pltpu.VMEM((c * 4,), F32),
            pltpu.VMEM((c * h,), F32),
            pltpu.VMEM((2, LANES), F32),
            pltpu.SemaphoreType.DMA,
            pltpu.SemaphoreType.DMA,
            pltpu.SemaphoreType.DMA,
        ),
    )
    def pass1(q_hbm, kt_hbm, g_hbm, ea_hbm, dst_hbm, src_hbm,
              logits_hbm, stats_hbm,
              dsti, srci, qrows, krows, grows, eav, lbuf, statv,
              sem0, sem1, sem2):
        wid = lax.axis_index("s") * NC + lax.axis_index("c")
        base = wid * ew
        iota = lax.iota(jnp.int32, LANES)

        def chunk(ci, carry):
            off = base + ci * c
            pltpu.sync_copy(dst_hbm.at[pl.ds(off, c)], dsti)
            pltpu.sync_copy(src_hbm.at[pl.ds(off, c)], srci)
            cq = pltpu.async_copy(q_hbm.at[dsti], qrows, sem0)
            ck = pltpu.async_copy(kt_hbm.at[srci], krows, sem1)
            cg = pltpu.async_copy(g_hbm.at[dsti], grows, sem2)
            pltpu.sync_copy(ea_hbm.at[pl.ds(off * 4, c * 4)], eav)
            cq.wait()
            ck.wait()
            cg.wait()

            def pair(j, carry2):
                m_r, z_r = carry2
                lv = jnp.zeros((LANES,), F32)
                for e2 in range(2):
                    eidx = 2 * j + e2
                    for h_ in range(h):
                        qv = qrows[eidx, pl.ds(h_ * dh, dh)]
                        kv = krows[eidx, pl.ds(h_ * dh, dh)]
                        lv = jnp.where(iota == (e2 * h + h_), jnp.sum(qv * kv), lv)
                    eav16 = eav[pl.ds((eidx // 4) * 16, 16)]
                    earep = _dg(eav16, (eidx % 4) * 4 + (iota & 3))
                    t0 = grows[eidx, pl.ds(0, 16)] * earep
                    t1 = grows[eidx, pl.ds(16, 16)] * earep
                    u0 = t0 + _dg(t0, iota ^ 1)
                    u0 = u0 + _dg(u0, iota ^ 2)
                    u1 = t1 + _dg(t1, iota ^ 1)
                    u1 = u1 + _dg(u1, iota ^ 2)
                    base4 = (iota & 3) * 4
                    w = jnp.where((iota & 7) < 4, _dg(u0, base4), _dg(u1, base4))
                    lv = lv + jnp.where((iota // 8) == e2, w, 0.0)
                lv = lv * scale
                lbuf[pl.ds(j * 16, 16)] = lv
                m_new = jnp.maximum(m_r, lv)
                z_new = z_r * jnp.exp(m_r - m_new) + jnp.exp(lv - m_new)
                return (m_new, z_new)

            carry = lax.fori_loop(0, c // 2, pair, carry)
            pltpu.sync_copy(lbuf, logits_hbm.at[pl.ds(off * h, c * h)])
            return carry

        m0 = jnp.full((LANES,), -3e38, F32)
        z0 = jnp.zeros((LANES,), F32)
        m_run, z_run = lax.fori_loop(0, ew // c, chunk, (m0, z0))
        statv[0, :] = m_run
        statv[1, :] = z_run
        pltpu.sync_copy(statv, stats_hbm.at[wid])

    return pass1


def _acc_rows_per_tile(n):
    return ((n + NS * 8 - 1) // (NS * 8)) * 8


def _build_pass2(n, e, d, h, c):
    ew = e // NW
    dh = d // h
    rpt = _acc_rows_per_tile(n)
    npad = rpt * NS
    mesh = plsc.VectorSubcoreMesh(
        core_axis_name="c", subcore_axis_name="s", num_cores=NC, num_subcores=NS
    )

    @functools.partial(
        pl.kernel,
        out_type=jax.ShapeDtypeStruct((NC, npad, d), F32),
        mesh=mesh,
        compiler_params=pltpu.CompilerParams(needs_layout_passes=False),
        scratch_types=(
            pltpu.VMEM((c,), jnp.int32),
            pltpu.VMEM((c,), jnp.int32),
            pltpu.VMEM((c, d), F32),
            pltpu.VMEM((c, d), F32),
            pltpu.VMEM((c * h,), F32),
            pltpu.VMEM((LANES,), F32),
            pltpu.VMEM_SHARED((npad, d), F32),
            pltpu.SemaphoreType.DMA,
        ),
    )
    def pass2(v_hbm, logits_hbm, dst_hbm, src_hbm, mrow_hbm, zero_hbm,
              part_hbm,
              dsti, srci, vrows, msg, lbuf, mbuf, accum, sem0):
        cid = lax.axis_index("c")
        sid = lax.axis_index("s")
        wid = sid * NC + cid
        base = wid * ew
        r0 = sid * rpt
        pltpu.sync_copy(zero_hbm.at[pl.ds(r0, rpt)], accum.at[pl.ds(r0, rpt)])
        pltpu.sync_copy(mrow_hbm, mbuf)
        plsc.subcore_barrier()
        mv = mbuf[...]

        def chunk(ci, _):
            off = base + ci * c
            pltpu.sync_copy(dst_hbm.at[pl.ds(off, c)], dsti)
            pltpu.sync_copy(src_hbm.at[pl.ds(off, c)], srci)
            cv = pltpu.async_copy(v_hbm.at[srci], vrows, sem0)
            pltpu.sync_copy(logits_hbm.at[pl.ds(off * h, c * h)], lbuf)
            cv.wait()

            def pair(j, _2):
                p = jnp.exp(lbuf[pl.ds(j * 16, 16)] - mv)
                for e2 in range(2):
                    eidx = 2 * j + e2
                    for h_ in range(h):
                        pv = _dg(p, jnp.zeros((16,), jnp.int32) + (e2 * h + h_))
                        msg[eidx, pl.ds(h_ * dh, dh)] = (
                            vrows[eidx, pl.ds(h_ * dh, dh)] * pv
                        )
                return 0

            lax.fori_loop(0, c // 2, pair, 0)
            pltpu.sync_copy(msg, accum.at[dsti], add=True)
            return 0

        lax.fori_loop(0, ew // c, chunk, 0)
        plsc.subcore_barrier()
        pltpu.sync_copy(accum.at[pl.ds(r0, rpt)], part_hbm.at[cid, pl.ds(r0, rpt)])

    return pass2


def _build_dense_first(n, d, bn):
    def body(x_ref, wc_ref, bc_ref, wem_ref, q_ref, k_ref, v_ref, g_ref):
        y = jnp.dot(x_ref[...], wc_ref[...], preferred_element_type=F32)
        y = y + bc_ref[...]
        q = y[:, :d]
        q_ref[...] = q
        k_ref[...] = y[:, d:2 * d]
        v_ref[...] = y[:, 2 * d:3 * d]
        g_ref[...] = jnp.dot(q, wem_ref[...], preferred_element_type=F32)

    g32 = d
    return pl.pallas_call(
        body,
        grid=(n // bn,),
        in_specs=[
            pl.BlockSpec((bn, d), lambda i: (i, 0)),
            pl.BlockSpec((d, 3 * d), lambda i: (0, 0)),
            pl.BlockSpec((1, 3 * d), lambda i: (0, 0)),
            pl.BlockSpec((d, g32), lambda i: (0, 0)),
        ],
        out_specs=(
            pl.BlockSpec((bn, d), lambda i: (i, 0)),
            pl.BlockSpec((bn, d), lambda i: (i, 0)),
            pl.BlockSpec((bn, d), lambda i: (i, 0)),
            pl.BlockSpec((bn, g32), lambda i: (i, 0)),
        ),
        out_shape=(
            jax.ShapeDtypeStruct((n, d), F32),
            jax.ShapeDtypeStruct((n, d), F32),
            jax.ShapeDtypeStruct((n, d), F32),
            jax.ShapeDtypeStruct((n, g32), F32),
        ),
    )


def _build_dense_mid(n, d, bn):
    def body(p0_ref, p1_ref, izr_ref, wo_ref, bo_ref, wc_ref, bc_ref, wem_ref,
             q_ref, k_ref, v_ref, g_ref):
        agg = (p0_ref[0] + p1_ref[0]) * izr_ref[...]
        xm = jnp.dot(agg, wo_ref[...], preferred_element_type=F32) + bo_ref[...]
        xm = jnp.maximum(xm, 0.0)
        y = jnp.dot(xm, wc_ref[...], preferred_element_type=F32) + bc_ref[...]
        q = y[:, :d]
        q_ref[...] = q
        k_ref[...] = y[:, d:2 * d]
        v_ref[...] = y[:, 2 * d:3 * d]
        g_ref[...] = jnp.dot(q, wem_ref[...], preferred_element_type=F32)

    g32 = d
    npad = _acc_rows_per_tile(n) * NS
    return pl.pallas_call(
        body,
        grid=(n // bn,),
        in_specs=[
            pl.BlockSpec((1, bn, d), lambda i: (0, i, 0)),
            pl.BlockSpec((1, bn, d), lambda i: (1, i, 0)),
            pl.BlockSpec((1, d), lambda i: (0, 0)),
            pl.BlockSpec((d, d), lambda i: (0, 0)),
            pl.BlockSpec((1, d), lambda i: (0, 0)),
            pl.BlockSpec((d, 3 * d), lambda i: (0, 0)),
            pl.BlockSpec((1, 3 * d), lambda i: (0, 0)),
            pl.BlockSpec((d, g32), lambda i: (0, 0)),
        ],
        out_specs=(
            pl.BlockSpec((bn, d), lambda i: (i, 0)),
            pl.BlockSpec((bn, d), lambda i: (i, 0)),
            pl.BlockSpec((bn, d), lambda i: (i, 0)),
            pl.BlockSpec((bn, g32), lambda i: (i, 0)),
        ),
        out_shape=(
            jax.ShapeDtypeStruct((n, d), F32),
            jax.ShapeDtypeStruct((n, d), F32),
            jax.ShapeDtypeStruct((n, d), F32),
            jax.ShapeDtypeStruct((n, g32), F32),
        ),
    )


def _build_dense_final(n, d, bn):
    def body(p0_ref, p1_ref, izr_ref, wo_ref, bo_ref, o_ref):
        agg = (p0_ref[0] + p1_ref[0]) * izr_ref[...]
        o_ref[...] = jnp.dot(agg, wo_ref[...], preferred_element_type=F32) + bo_ref[...]

    return pl.pallas_call(
        body,
        grid=(n // bn,),
        in_specs=[
            pl.BlockSpec((1, bn, d), lambda i: (0, i, 0)),
            pl.BlockSpec((1, bn, d), lambda i: (1, i, 0)),
            pl.BlockSpec((1, d), lambda i: (0, 0)),
            pl.BlockSpec((d, d), lambda i: (0, 0)),
            pl.BlockSpec((1, d), lambda i: (0, 0)),
        ],
        out_specs=pl.BlockSpec((bn, d), lambda i: (i, 0)),
        out_shape=jax.ShapeDtypeStruct((n, d), F32),
    )


def kernel(x, edge_index, edge_attr, Wq, Wk, Wv, We, be, Wo, bo):
    n, d = x.shape
    e = edge_index.shape[1]
    nl = Wq.shape[0]
    ed = edge_attr.shape[1]
    h = 8
    dh = d // h
    c = 80
    bn = 2000

    src = edge_index[0]
    dst = edge_index[1]
    ea_flat = edge_attr.reshape(-1)
    npad = _acc_rows_per_tile(n) * NS
    zeros_nd = jnp.zeros((npad, d), F32)

    mask = (jnp.arange(d)[:, None] // dh == jnp.arange(4 * h)[None, :] // 4)
    WeM = jnp.tile(jnp.transpose(We, (0, 2, 1)), (1, 1, h)) * mask[None].astype(F32)
    WeM = jnp.concatenate([WeM, jnp.zeros((nl, d, d - 4 * h), F32)], axis=2)
    Wcat = jnp.concatenate([Wq, Wk, Wv], axis=2)
    zcol = jnp.zeros((nl, d), F32)
    bias = jnp.concatenate([zcol, be, zcol], axis=1)[:, None, :]

    pass1 = _build_pass1(n, e, d, h, c)
    pass2 = _build_pass2(n, e, d, h, c)
    dense_first = _build_dense_first(n, d, bn)
    dense_mid = _build_dense_mid(n, d, bn)
    dense_final = _build_dense_final(n, d, bn)

    qkv = dense_first(x, Wcat[0], bias[0], WeM[0])
    out = None
    for l in range(nl):
        q, kt, v, g = qkv
        logits, stats = pass1(q, kt, g, ea_flat, dst, src)
        m_w = stats[:, 0, :]
        z_w = stats[:, 1, :]
        m16 = jnp.max(m_w, axis=0)
        m8 = jnp.maximum(m16[:8], m16[8:])
        mrow = jnp.concatenate([m8, m8])
        z16 = jnp.sum(z_w * jnp.exp(m_w - mrow[None, :]), axis=0)
        z8 = z16[:8] + z16[8:]
        izr = jnp.repeat(1.0 / z8, dh)[None, :]
        part = pass2(v, logits, dst, src, mrow, zeros_nd)
        if l < nl - 1:
            qkv = dense_mid(part, part, izr, Wo[l], bo[l][None, :],
                            Wcat[l + 1], bias[l + 1], WeM[l + 1])
        else:
            out = dense_final(part, part, izr, Wo[l], bo[l][None, :])
    return out
